# baseline (speedup 1.0000x reference)
import functools

import jax
import jax.numpy as jnp
from jax import lax
from jax.experimental import pallas as pl
from jax.experimental.pallas import tpu as pltpu
from jax.experimental.pallas import tpu_sc as plsc

_N_USER = 10000
_N_ITEM = 10000
_D = 128
_E = 320000

_NUM_TILES = 16
_CHUNK = 128
_EPT = 20480
_NCHUNKS = _EPT // _CHUNK
_EPAD = _EPT * _NUM_TILES
_G = 8
_NPAIRS = _NCHUNKS // 2
_N_PAD = 10240
_ROWS_PER_TILE = _N_PAD // _NUM_TILES


def _sc_aggregate(table, idx4, zeros):
    mesh = plsc.VectorSubcoreMesh(core_axis_name="c", subcore_axis_name="s")

    @functools.partial(
        pl.kernel,
        out_type=jax.ShapeDtypeStruct((2, _N_PAD, _D), jnp.float32),
        mesh=mesh,
        scratch_types=[
            pltpu.VMEM((2, _G, 2, _CHUNK), jnp.int32),
            pltpu.VMEM((2, _CHUNK, _D), jnp.float32),
            pltpu.VMEM_SHARED((_N_PAD, _D), jnp.float32),
            pltpu.SemaphoreType.DMA,
            pltpu.SemaphoreType.DMA((2,)),
            pltpu.SemaphoreType.DMA((2,)),
        ],
    )
    def agg_kernel(table_hbm, idx_hbm, zeros_hbm, out_hbm,
                   idx_v, rows_v, acc_sh, isem, gsem, ssem):
        c = lax.axis_index("c")
        s = lax.axis_index("s")
        w = c * _NUM_TILES + s
        rbase = s * _ROWS_PER_TILE

        def idx_load(g):
            return pltpu.make_async_copy(
                idx_hbm.at[w, pl.ds(g * _G, _G)], idx_v.at[g % 2], isem)

        def gather(i, j):
            src_row = idx_v.at[(i // _G) % 2, i % _G, 0]
            return pltpu.make_async_copy(
                table_hbm.at[src_row], rows_v.at[j], gsem.at[j])

        def scatter(i, j):
            dst_row = idx_v.at[(i // _G) % 2, i % _G, 1]
            return pltpu.make_async_copy(
                rows_v.at[j], acc_sh.at[dst_row], ssem.at[j])

        idx_load(0).start()
        pltpu.sync_copy(zeros_hbm.at[pl.ds(rbase, _ROWS_PER_TILE)],
                        acc_sh.at[pl.ds(rbase, _ROWS_PER_TILE)])
        plsc.subcore_barrier()
        idx_load(0).wait()
        idx_load(1).start()
        gather(0, 0).start()

        @pl.loop(0, _NPAIRS)
        def _(p):
            @pl.when(((p & 3) == 3) & (p < (_NPAIRS - 4)))
            def _():
                idx_load(p // 4 + 1).wait()

            for j in range(2):
                i = 2 * p + j
                jn = (j + 1) % 2

                @pl.when(i >= 1)
                def _():
                    scatter(i - 1, jn).wait()

                @pl.when(i + 1 < _NCHUNKS)
                def _():
                    gather(i + 1, jn).start()

                gather(i, j).wait()
                scatter(i, j).start(add=True)

            @pl.when(((p & 3) == 0) & (p > 0) & (p < (_NPAIRS - 4)))
            def _():
                idx_load(p // 4 + 1).start()

        scatter(_NCHUNKS - 1, (_NCHUNKS - 1) % 2).wait()

        plsc.subcore_barrier()
        pltpu.sync_copy(acc_sh.at[pl.ds(rbase, _ROWS_PER_TILE)],
                        out_hbm.at[c, pl.ds(rbase, _ROWS_PER_TILE)])

    return agg_kernel(table, idx4, zeros)


def _affine_kernel(agg0_ref, agg1_ref, xi_ref, xu_ref,
                   wm0_ref, wr0_ref, b0_ref, wm1_ref, wr1_ref, b1_ref,
                   oi_ref, ou_ref):
    oi_ref[...] = (
        jnp.dot(agg0_ref[0], wm0_ref[...], preferred_element_type=jnp.float32)
        + jnp.dot(xi_ref[...], wr0_ref[...], preferred_element_type=jnp.float32)
        + b0_ref[...]
    )
    ou_ref[...] = (
        jnp.dot(agg1_ref[0], wm1_ref[...], preferred_element_type=jnp.float32)
        + jnp.dot(xu_ref[...], wr1_ref[...], preferred_element_type=jnp.float32)
        + b1_ref[...]
    )


def _tc_epilogue(agg, x_item, x_user, wm0, wr0, b0, wm1, wr1, b1):
    n = x_item.shape[0]
    blk = 2000
    grid = (n // blk,)
    row_spec = pl.BlockSpec((blk, _D), lambda i: (i, 0))
    w_spec = pl.BlockSpec((_D, _D), lambda i: (0, 0))
    b_spec = pl.BlockSpec((1, _D), lambda i: (0, 0))
    return pl.pallas_call(
        _affine_kernel,
        grid=grid,
        in_specs=[
            pl.BlockSpec((1, blk, _D), lambda i: (0, i, 0)),
            pl.BlockSpec((1, blk, _D), lambda i: (1, i, 0)),
            row_spec, row_spec,
            w_spec, w_spec, b_spec,
            w_spec, w_spec, b_spec,
        ],
        out_specs=[row_spec, row_spec],
        out_shape=[
            jax.ShapeDtypeStruct((n, _D), jnp.float32),
            jax.ShapeDtypeStruct((n, _D), jnp.float32),
        ],
    )(agg, agg, x_item, x_user, wm0, wr0, b0.reshape(1, _D),
      wm1, wr1, b1.reshape(1, _D))


def kernel(x_user, x_item, edge_index_u2i, edge_index_i2u,
           W_msg_u2i, W_root_u2i, b_u2i,
           W_msg_i2u, W_root_i2u, b_i2u):
    table = jnp.concatenate([x_user, x_item], axis=0)
    npad_e = _EPAD - _E
    src_pad = jnp.zeros((npad_e,), jnp.int32)
    dst_pad = jnp.full((npad_e,), _N_PAD - 1, jnp.int32)
    src3 = jnp.concatenate([
        edge_index_u2i[0].astype(jnp.int32), src_pad,
        edge_index_i2u[0].astype(jnp.int32) + _N_USER, src_pad,
    ]).reshape(2 * _NUM_TILES, _NCHUNKS, _CHUNK)
    dst3 = jnp.concatenate([
        edge_index_u2i[1].astype(jnp.int32), dst_pad,
        edge_index_i2u[1].astype(jnp.int32), dst_pad,
    ]).reshape(2 * _NUM_TILES, _NCHUNKS, _CHUNK)
    idx4 = jnp.stack([src3, dst3], axis=2)
    zeros = jnp.zeros((_N_PAD, _D), jnp.float32)
    agg = _sc_aggregate(table, idx4, zeros)
    out_item, out_user = _tc_epilogue(
        agg, x_item, x_user,
        W_msg_u2i, W_root_u2i, b_u2i,
        W_msg_i2u, W_root_i2u, b_i2u)
    return (out_user, out_item)
